# baseline (speedup 1.0000x reference)
import functools
import jax
import jax.numpy as jnp
from jax import lax
from jax.experimental import pallas as pl
from jax.experimental.pallas import tpu as pltpu
from jax.experimental.pallas import tpu_sc as plsc

N = 10000
D = 128
E = 320000
G = 64
O = 64

NC = 2
NS = 16
NW = NC * NS

NP = 10240
K = 128
C = 80
EP = NW * C * K
RPT = NP // NS

_mesh = plsc.VectorSubcoreMesh(core_axis_name="c", subcore_axis_name="s")


def _deg_body(dst_hbm, zeros_hbm, out_hbm, dst_v, ones_v, acc):
    cid = lax.axis_index("c")
    sid = lax.axis_index("s")
    wid = sid * NC + cid

    @pl.when(sid == 0)
    def _():
        pltpu.sync_copy(zeros_hbm, acc)
    for j in range(K // 16):
        ones_v[pl.ds(j * 16, 16)] = jnp.ones((16,), jnp.float32)
    plsc.subcore_barrier()

    pltpu.sync_copy(dst_hbm.at[wid], dst_v)

    def chunk(c, carry):
        pltpu.sync_copy(ones_v, acc.at[dst_v.at[c]], add=True)
        return carry
    lax.fori_loop(0, C, chunk, 0)

    plsc.subcore_barrier()
    pltpu.sync_copy(acc.at[pl.ds(sid * RPT, RPT)],
                    out_hbm.at[cid, pl.ds(sid * RPT, RPT)])


_deg_call = pl.kernel(
    _deg_body,
    out_type=jax.ShapeDtypeStruct((NC, NP), jnp.float32),
    mesh=_mesh,
    scratch_types=[
        pltpu.VMEM((C, K), jnp.int32),
        pltpu.VMEM((K,), jnp.float32),
        pltpu.VMEM_SHARED((NP,), jnp.float32),
    ],
)


def _agg_body(m_hbm, src_hbm, dst_hbm, zeros_hbm, out_hbm,
              si0, si1, di0, di1, rows0, rows1, is0, is1, g0, g1, acc):
    cid = lax.axis_index("c")
    sid = lax.axis_index("s")
    wid = sid * NC + cid

    def fetch(c, si, di, sem):
        pltpu.async_copy(src_hbm.at[wid, c], si, sem)
        pltpu.async_copy(dst_hbm.at[wid, c], di, sem)

    def fetch_wait(c, si, di, sem):
        pltpu.make_async_copy(src_hbm.at[wid, c], si, sem).wait()
        pltpu.make_async_copy(dst_hbm.at[wid, c], di, sem).wait()

    fetch(0, si0, di0, is0)
    fetch(1, si1, di1, is1)
    fetch_wait(0, si0, di0, is0)
    pltpu.async_copy(m_hbm.at[si0], rows0, g0)
    pltpu.sync_copy(zeros_hbm.at[pl.ds(sid * RPT, RPT)],
                    acc.at[pl.ds(sid * RPT, RPT)])
    plsc.subcore_barrier()

    C2 = C // 2

    def chunk(j, carry):
        c0 = 2 * j
        fetch_wait(c0 + 1, si1, di1, is1)
        pltpu.async_copy(m_hbm.at[si1], rows1, g1)
        pltpu.make_async_copy(m_hbm.at[si0], rows0, g0).wait()
        pltpu.sync_copy(rows0, acc.at[di0], add=True)

        @pl.when(j + 1 < C2)
        def _():
            fetch(c0 + 2, si0, di0, is0)

        pltpu.make_async_copy(m_hbm.at[si1], rows1, g1).wait()
        pltpu.sync_copy(rows1, acc.at[di1], add=True)

        @pl.when(j + 1 < C2)
        def _():
            fetch_wait(c0 + 2, si0, di0, is0)
            pltpu.async_copy(m_hbm.at[si0], rows0, g0)
            fetch(c0 + 3, si1, di1, is1)
        return carry
    lax.fori_loop(0, C2, chunk, 0)

    plsc.subcore_barrier()
    pltpu.sync_copy(acc.at[pl.ds(sid * RPT, RPT)],
                    out_hbm.at[cid, pl.ds(sid * RPT, RPT)])


_agg_call = pl.kernel(
    _agg_body,
    out_type=jax.ShapeDtypeStruct((NC, NP, D), jnp.float32),
    mesh=_mesh,
    scratch_types=[
        pltpu.VMEM((K,), jnp.int32),
        pltpu.VMEM((K,), jnp.int32),
        pltpu.VMEM((K,), jnp.int32),
        pltpu.VMEM((K,), jnp.int32),
        pltpu.VMEM((K, D), jnp.float32),
        pltpu.VMEM((K, D), jnp.float32),
        pltpu.SemaphoreType.DMA,
        pltpu.SemaphoreType.DMA,
        pltpu.SemaphoreType.DMA,
        pltpu.SemaphoreType.DMA,
        pltpu.VMEM_SHARED((NP, D), jnp.float32),
    ],
)


R = 1024
NB = NP // R


def _mm_body(x_ref, w1_ref, m1r_ref):
    m1r_ref[...] = jnp.dot(x_ref[...], w1_ref[...],
                           preferred_element_type=jnp.float32)


def _mm_call(x_pad, W1):
    return pl.pallas_call(
        _mm_body,
        grid=(NB,),
        in_specs=[
            pl.BlockSpec((R, D), lambda i: (i, 0)),
            pl.BlockSpec((D, D), lambda i: (0, 0)),
        ],
        out_specs=pl.BlockSpec((R, D), lambda i: (i, 0)),
        out_shape=jax.ShapeDtypeStruct((NP, D), jnp.float32),
    )(x_pad, W1)


def _prep_body(deg_ref, m1r_ref, m1_ref, dinv_ref):
    i = pl.program_id(0)
    deg = deg_ref[0] + deg_ref[1] + 1.0
    rows = (jax.lax.broadcasted_iota(jnp.int32, (R, 1), 0) + i * R)
    valid = jnp.where(rows < N, 1.0, 0.0)
    dinv = jax.lax.rsqrt(deg)[:, None] * valid
    dinv_ref[...] = dinv
    m1_ref[...] = m1r_ref[...] * dinv


def _prep_call(deg2, m1r):
    return pl.pallas_call(
        _prep_body,
        grid=(NB,),
        in_specs=[
            pl.BlockSpec((2, R), lambda i: (0, i)),
            pl.BlockSpec((R, D), lambda i: (i, 0)),
        ],
        out_specs=[
            pl.BlockSpec((R, D), lambda i: (i, 0)),
            pl.BlockSpec((R, 1), lambda i: (i, 0)),
        ],
        out_shape=[
            jax.ShapeDtypeStruct((NP, D), jnp.float32),
            jax.ShapeDtypeStruct((NP, 1), jnp.float32),
        ],
    )(deg2, m1r)


def _mid_body(a0_ref, a1_ref, m1_ref, dinv_ref, b1_ref, w2_ref, m2_ref):
    dinv = dinv_ref[...]
    h1 = jnp.maximum((a0_ref[...] + a1_ref[...] + m1_ref[...]) * dinv
                     + b1_ref[...], 0.0)
    m2_ref[...] = jnp.dot(h1, w2_ref[...],
                          preferred_element_type=jnp.float32) * dinv


def _mid_call(a0, a1, m1p, dinv, b1r, W2):
    return pl.pallas_call(
        _mid_body,
        grid=(NB,),
        in_specs=[
            pl.BlockSpec((R, D), lambda i: (i, 0)),
            pl.BlockSpec((R, D), lambda i: (i, 0)),
            pl.BlockSpec((R, D), lambda i: (i, 0)),
            pl.BlockSpec((R, 1), lambda i: (i, 0)),
            pl.BlockSpec((1, D), lambda i: (0, 0)),
            pl.BlockSpec((D, D), lambda i: (0, 0)),
        ],
        out_specs=pl.BlockSpec((R, D), lambda i: (i, 0)),
        out_shape=jax.ShapeDtypeStruct((NP, D), jnp.float32),
    )(a0, a1, m1p, dinv, b1r, W2)


def _final_body(a0_ref, a1_ref, m2_ref, dinv_ref, b2_ref, batch_ref,
                wf1_ref, bf1_ref, wf2_ref, bf2_ref, out_ref,
                pooled_acc, cnt_acc):
    i = pl.program_id(0)

    @pl.when(i == 0)
    def _():
        pooled_acc[...] = jnp.zeros((G, D), jnp.float32)
        cnt_acc[...] = jnp.zeros((G, 1), jnp.float32)

    h2 = jnp.maximum((a0_ref[...] + a1_ref[...] + m2_ref[...]) * dinv_ref[...]
                     + b2_ref[...], 0.0)
    b_row = batch_ref[...].reshape(1, R)
    gids = jax.lax.broadcasted_iota(jnp.int32, (G, R), 0).astype(jnp.float32)
    mask = jnp.where(gids == b_row, 1.0, 0.0)
    pooled_acc[...] += jnp.dot(mask, h2, preferred_element_type=jnp.float32)
    cnt_acc[...] += jnp.sum(mask, axis=1, keepdims=True)

    @pl.when(i == NB - 1)
    def _():
        pooled = pooled_acc[...] / jnp.maximum(cnt_acc[...], 1.0)
        z = jnp.maximum(jnp.dot(pooled, wf1_ref[...],
                                preferred_element_type=jnp.float32)
                        + bf1_ref[...], 0.0)
        out_ref[...] = jnp.dot(z, wf2_ref[...],
                               preferred_element_type=jnp.float32) + bf2_ref[...]


def _final_call(a0, a1, m2p, dinv, b2r, batch3, Wf1, bf1r, Wf2, bf2r):
    return pl.pallas_call(
        _final_body,
        grid=(NB,),
        in_specs=[
            pl.BlockSpec((R, D), lambda i: (i, 0)),
            pl.BlockSpec((R, D), lambda i: (i, 0)),
            pl.BlockSpec((R, D), lambda i: (i, 0)),
            pl.BlockSpec((R, 1), lambda i: (i, 0)),
            pl.BlockSpec((1, D), lambda i: (0, 0)),
            pl.BlockSpec((1, 1, R), lambda i: (i, 0, 0)),
            pl.BlockSpec((D, D // 2), lambda i: (0, 0)),
            pl.BlockSpec((1, D // 2), lambda i: (0, 0)),
            pl.BlockSpec((D // 2, O), lambda i: (0, 0)),
            pl.BlockSpec((1, O), lambda i: (0, 0)),
        ],
        out_specs=pl.BlockSpec((G, O), lambda i: (0, 0)),
        out_shape=jax.ShapeDtypeStruct((G, O), jnp.float32),
        scratch_shapes=[
            pltpu.VMEM((G, D), jnp.float32),
            pltpu.VMEM((G, 1), jnp.float32),
        ],
    )(a0, a1, m2p, dinv, b2r, batch3, Wf1, bf1r, Wf2, bf2r)


@jax.jit
def kernel(x, edge_index, batch, W1, b1, W2, b2, Wf1, bf1, Wf2, bf2):
    x_pad = jnp.zeros((NP, D), x.dtype).at[:N].set(x)
    npad = EP - E
    pad_idx = (N + (jnp.arange(npad, dtype=jnp.int32) % (NP - N))).astype(jnp.int32)
    src = jnp.concatenate([edge_index[0], pad_idx]).reshape(NW, C, K)
    dst = jnp.concatenate([edge_index[1], pad_idx]).reshape(NW, C, K)

    zeros1 = jnp.zeros((NP,), jnp.float32)
    zerosD = jnp.zeros((NP, D), jnp.float32)

    m1r = _mm_call(x_pad, W1)
    deg2 = _deg_call(dst, zeros1)

    m1p, dinv = _prep_call(deg2, m1r)

    agg1 = _agg_call(m1p, src, dst, zerosD)
    m2p = _mid_call(agg1[0], agg1[1], m1p, dinv, b1.reshape(1, D), W2)

    agg2 = _agg_call(m2p, src, dst, zerosD)

    batch3 = jnp.concatenate(
        [batch.astype(jnp.float32), jnp.full((NP - N,), float(G), jnp.float32)]
    ).reshape(NB, 1, R)
    return _final_call(agg2[0], agg2[1], m2p, dinv, b2.reshape(1, D),
                       batch3, Wf1, bf1.reshape(1, D // 2), Wf2,
                       bf2.reshape(1, O))


# speedup vs baseline: 1.0103x; 1.0103x over previous
import functools
import jax
import jax.numpy as jnp
from jax import lax
from jax.experimental import pallas as pl
from jax.experimental.pallas import tpu as pltpu
from jax.experimental.pallas import tpu_sc as plsc

N = 10000
D = 128
E = 320000
G = 64
O = 64

NC = 2
NS = 16
NW = NC * NS

NP = 10240
K = 128
C = 80
EP = NW * C * K
RPT = NP // NS

_mesh = plsc.VectorSubcoreMesh(core_axis_name="c", subcore_axis_name="s")


def _deg_body(dst_hbm, zeros_hbm, out_hbm, dst_v, ones_v, acc):
    cid = lax.axis_index("c")
    sid = lax.axis_index("s")
    wid = sid * NC + cid

    @pl.when(sid == 0)
    def _():
        pltpu.sync_copy(zeros_hbm, acc)
    for j in range(K // 16):
        ones_v[pl.ds(j * 16, 16)] = jnp.ones((16,), jnp.float32)
    plsc.subcore_barrier()

    pltpu.sync_copy(dst_hbm.at[wid], dst_v)

    def chunk(c, carry):
        pltpu.sync_copy(ones_v, acc.at[dst_v.at[c]], add=True)
        return carry
    lax.fori_loop(0, C, chunk, 0)

    plsc.subcore_barrier()
    pltpu.sync_copy(acc.at[pl.ds(sid * RPT, RPT)],
                    out_hbm.at[cid, pl.ds(sid * RPT, RPT)])


_deg_call = pl.kernel(
    _deg_body,
    out_type=jax.ShapeDtypeStruct((NC, NP), jnp.float32),
    mesh=_mesh,
    scratch_types=[
        pltpu.VMEM((C, K), jnp.int32),
        pltpu.VMEM((K,), jnp.float32),
        pltpu.VMEM_SHARED((NP,), jnp.float32),
    ],
)


def _agg_body(m_hbm, src_hbm, dst_hbm, zeros_hbm, out_hbm,
              si0, si1, di0, di1, rows0, rows1, is0, is1, g0, g1, ss0, ss1, acc):
    cid = lax.axis_index("c")
    sid = lax.axis_index("s")
    wid = sid * NC + cid

    def fetch(c, si, di, sem):
        pltpu.async_copy(src_hbm.at[wid, c], si, sem)
        pltpu.async_copy(dst_hbm.at[wid, c], di, sem)

    def fetch_wait(c, si, di, sem):
        pltpu.make_async_copy(src_hbm.at[wid, c], si, sem).wait()
        pltpu.make_async_copy(dst_hbm.at[wid, c], di, sem).wait()

    fetch(0, si0, di0, is0)
    fetch(1, si1, di1, is1)
    fetch_wait(0, si0, di0, is0)
    pltpu.async_copy(m_hbm.at[si0], rows0, g0)
    pltpu.sync_copy(zeros_hbm.at[pl.ds(sid * RPT, RPT)],
                    acc.at[pl.ds(sid * RPT, RPT)])
    plsc.subcore_barrier()

    C2 = C // 2

    def chunk(j, carry):
        c0 = 2 * j
        fetch_wait(c0 + 1, si1, di1, is1)
        pltpu.async_copy(m_hbm.at[si1], rows1, g1)
        pltpu.make_async_copy(m_hbm.at[si0], rows0, g0).wait()
        pltpu.async_copy(rows0, acc.at[di0], ss0, add=True)
        pltpu.make_async_copy(m_hbm.at[si1], rows1, g1).wait()
        pltpu.async_copy(rows1, acc.at[di1], ss1, add=True)

        @pl.when(j + 1 < C2)
        def _():
            pltpu.make_async_copy(rows0, acc.at[di0], ss0).wait()
            fetch(c0 + 2, si0, di0, is0)
            fetch_wait(c0 + 2, si0, di0, is0)
            pltpu.async_copy(m_hbm.at[si0], rows0, g0)
            pltpu.make_async_copy(rows1, acc.at[di1], ss1).wait()
            fetch(c0 + 3, si1, di1, is1)
        return carry
    lax.fori_loop(0, C2, chunk, 0)

    pltpu.make_async_copy(rows0, acc.at[di0], ss0).wait()
    pltpu.make_async_copy(rows1, acc.at[di1], ss1).wait()
    plsc.subcore_barrier()
    pltpu.sync_copy(acc.at[pl.ds(sid * RPT, RPT)],
                    out_hbm.at[cid, pl.ds(sid * RPT, RPT)])


_agg_call = pl.kernel(
    _agg_body,
    out_type=jax.ShapeDtypeStruct((NC, NP, D), jnp.float32),
    mesh=_mesh,
    scratch_types=[
        pltpu.VMEM((K,), jnp.int32),
        pltpu.VMEM((K,), jnp.int32),
        pltpu.VMEM((K,), jnp.int32),
        pltpu.VMEM((K,), jnp.int32),
        pltpu.VMEM((K, D), jnp.float32),
        pltpu.VMEM((K, D), jnp.float32),
        pltpu.SemaphoreType.DMA,
        pltpu.SemaphoreType.DMA,
        pltpu.SemaphoreType.DMA,
        pltpu.SemaphoreType.DMA,
        pltpu.SemaphoreType.DMA,
        pltpu.SemaphoreType.DMA,
        pltpu.VMEM_SHARED((NP, D), jnp.float32),
    ],
)


R = 1024
NB = NP // R


def _mm_body(x_ref, w1_ref, m1r_ref):
    m1r_ref[...] = jnp.dot(x_ref[...], w1_ref[...],
                           preferred_element_type=jnp.float32)


def _mm_call(x_pad, W1):
    return pl.pallas_call(
        _mm_body,
        grid=(NB,),
        in_specs=[
            pl.BlockSpec((R, D), lambda i: (i, 0)),
            pl.BlockSpec((D, D), lambda i: (0, 0)),
        ],
        out_specs=pl.BlockSpec((R, D), lambda i: (i, 0)),
        out_shape=jax.ShapeDtypeStruct((NP, D), jnp.float32),
    )(x_pad, W1)


def _prep_body(deg_ref, m1r_ref, m1_ref, dinv_ref):
    i = pl.program_id(0)
    deg = deg_ref[0] + deg_ref[1] + 1.0
    rows = (jax.lax.broadcasted_iota(jnp.int32, (R, 1), 0) + i * R)
    valid = jnp.where(rows < N, 1.0, 0.0)
    dinv = jax.lax.rsqrt(deg)[:, None] * valid
    dinv_ref[...] = dinv
    m1_ref[...] = m1r_ref[...] * dinv


def _prep_call(deg2, m1r):
    return pl.pallas_call(
        _prep_body,
        grid=(NB,),
        in_specs=[
            pl.BlockSpec((2, R), lambda i: (0, i)),
            pl.BlockSpec((R, D), lambda i: (i, 0)),
        ],
        out_specs=[
            pl.BlockSpec((R, D), lambda i: (i, 0)),
            pl.BlockSpec((R, 1), lambda i: (i, 0)),
        ],
        out_shape=[
            jax.ShapeDtypeStruct((NP, D), jnp.float32),
            jax.ShapeDtypeStruct((NP, 1), jnp.float32),
        ],
    )(deg2, m1r)


def _mid_body(a0_ref, a1_ref, m1_ref, dinv_ref, b1_ref, w2_ref, m2_ref):
    dinv = dinv_ref[...]
    h1 = jnp.maximum((a0_ref[...] + a1_ref[...] + m1_ref[...]) * dinv
                     + b1_ref[...], 0.0)
    m2_ref[...] = jnp.dot(h1, w2_ref[...],
                          preferred_element_type=jnp.float32) * dinv


def _mid_call(a0, a1, m1p, dinv, b1r, W2):
    return pl.pallas_call(
        _mid_body,
        grid=(NB,),
        in_specs=[
            pl.BlockSpec((R, D), lambda i: (i, 0)),
            pl.BlockSpec((R, D), lambda i: (i, 0)),
            pl.BlockSpec((R, D), lambda i: (i, 0)),
            pl.BlockSpec((R, 1), lambda i: (i, 0)),
            pl.BlockSpec((1, D), lambda i: (0, 0)),
            pl.BlockSpec((D, D), lambda i: (0, 0)),
        ],
        out_specs=pl.BlockSpec((R, D), lambda i: (i, 0)),
        out_shape=jax.ShapeDtypeStruct((NP, D), jnp.float32),
    )(a0, a1, m1p, dinv, b1r, W2)


def _final_body(a0_ref, a1_ref, m2_ref, dinv_ref, b2_ref, batch_ref,
                wf1_ref, bf1_ref, wf2_ref, bf2_ref, out_ref,
                pooled_acc, cnt_acc):
    i = pl.program_id(0)

    @pl.when(i == 0)
    def _():
        pooled_acc[...] = jnp.zeros((G, D), jnp.float32)
        cnt_acc[...] = jnp.zeros((G, 1), jnp.float32)

    h2 = jnp.maximum((a0_ref[...] + a1_ref[...] + m2_ref[...]) * dinv_ref[...]
                     + b2_ref[...], 0.0)
    b_row = batch_ref[...].reshape(1, R)
    gids = jax.lax.broadcasted_iota(jnp.int32, (G, R), 0).astype(jnp.float32)
    mask = jnp.where(gids == b_row, 1.0, 0.0)
    pooled_acc[...] += jnp.dot(mask, h2, preferred_element_type=jnp.float32)
    cnt_acc[...] += jnp.sum(mask, axis=1, keepdims=True)

    @pl.when(i == NB - 1)
    def _():
        pooled = pooled_acc[...] / jnp.maximum(cnt_acc[...], 1.0)
        z = jnp.maximum(jnp.dot(pooled, wf1_ref[...],
                                preferred_element_type=jnp.float32)
                        + bf1_ref[...], 0.0)
        out_ref[...] = jnp.dot(z, wf2_ref[...],
                               preferred_element_type=jnp.float32) + bf2_ref[...]


def _final_call(a0, a1, m2p, dinv, b2r, batch3, Wf1, bf1r, Wf2, bf2r):
    return pl.pallas_call(
        _final_body,
        grid=(NB,),
        in_specs=[
            pl.BlockSpec((R, D), lambda i: (i, 0)),
            pl.BlockSpec((R, D), lambda i: (i, 0)),
            pl.BlockSpec((R, D), lambda i: (i, 0)),
            pl.BlockSpec((R, 1), lambda i: (i, 0)),
            pl.BlockSpec((1, D), lambda i: (0, 0)),
            pl.BlockSpec((1, 1, R), lambda i: (i, 0, 0)),
            pl.BlockSpec((D, D // 2), lambda i: (0, 0)),
            pl.BlockSpec((1, D // 2), lambda i: (0, 0)),
            pl.BlockSpec((D // 2, O), lambda i: (0, 0)),
            pl.BlockSpec((1, O), lambda i: (0, 0)),
        ],
        out_specs=pl.BlockSpec((G, O), lambda i: (0, 0)),
        out_shape=jax.ShapeDtypeStruct((G, O), jnp.float32),
        scratch_shapes=[
            pltpu.VMEM((G, D), jnp.float32),
            pltpu.VMEM((G, 1), jnp.float32),
        ],
    )(a0, a1, m2p, dinv, b2r, batch3, Wf1, bf1r, Wf2, bf2r)


@jax.jit
def kernel(x, edge_index, batch, W1, b1, W2, b2, Wf1, bf1, Wf2, bf2):
    x_pad = jnp.zeros((NP, D), x.dtype).at[:N].set(x)
    npad = EP - E
    pad_idx = (N + (jnp.arange(npad, dtype=jnp.int32) % (NP - N))).astype(jnp.int32)
    src = jnp.concatenate([edge_index[0], pad_idx]).reshape(NW, C, K)
    dst = jnp.concatenate([edge_index[1], pad_idx]).reshape(NW, C, K)

    zeros1 = jnp.zeros((NP,), jnp.float32)
    zerosD = jnp.zeros((NP, D), jnp.float32)

    m1r = _mm_call(x_pad, W1)
    deg2 = _deg_call(dst, zeros1)

    m1p, dinv = _prep_call(deg2, m1r)

    agg1 = _agg_call(m1p, src, dst, zerosD)
    m2p = _mid_call(agg1[0], agg1[1], m1p, dinv, b1.reshape(1, D), W2)

    agg2 = _agg_call(m2p, src, dst, zerosD)

    batch3 = jnp.concatenate(
        [batch.astype(jnp.float32), jnp.full((NP - N,), float(G), jnp.float32)]
    ).reshape(NB, 1, R)
    return _final_call(agg2[0], agg2[1], m2p, dinv, b2.reshape(1, D),
                       batch3, Wf1, bf1.reshape(1, D // 2), Wf2,
                       bf2.reshape(1, O))
